# Initial kernel scaffold; baseline (speedup 1.0000x reference)
import jax
import jax.numpy as jnp
from jax.experimental import pallas as pl


def kernel(sample, entity_embedding, relation_embedding):
    raise NotImplementedError("write your pallas kernel here")


# speedup vs baseline: 1.1996x; 1.1996x over previous
import functools

import jax
import jax.numpy as jnp
from jax import lax
from jax.experimental import pallas as pl
from jax.experimental.pallas import tpu as pltpu
from jax.experimental.pallas import tpu_sc as plsc

GAMMA = 12.0
B = 4096
D = 128
NC = 2
NS = 16
NW = NC * NS
BPW = B // NW
LANES = 16


def _sc_body(hidx_hbm, ridx_hbm, tidx_hbm, ent_hbm, rel_hbm, out_hbm,
             hidx_v, ridx_v, tidx_v, h_v, r_v, t_v, out_v,
             sem_h, sem_r, sem_t):
    wid = lax.axis_index("s") * NC + lax.axis_index("c")
    base = wid * BPW
    pltpu.sync_copy(hidx_hbm.at[pl.ds(base, BPW)], hidx_v)
    pltpu.sync_copy(ridx_hbm.at[pl.ds(base, BPW)], ridx_v)
    pltpu.sync_copy(tidx_hbm.at[pl.ds(base, BPW)], tidx_v)
    ch = pltpu.async_copy(ent_hbm.at[hidx_v], h_v, sem_h)
    cr = pltpu.async_copy(rel_hbm.at[ridx_v], r_v, sem_r)
    ct = pltpu.async_copy(ent_hbm.at[tidx_v], t_v, sem_t)
    ch.wait()
    cr.wait()
    ct.wait()

    lane = lax.iota(jnp.int32, LANES)

    def group_body(g, carry):
        scores = jnp.full((LANES,), GAMMA, jnp.float32)
        for l in range(LANES):
            i = g * LANES + l
            acc = jnp.zeros((LANES,), jnp.float32)
            for j in range(D // LANES):
                hv = h_v[i, pl.ds(j * LANES, LANES)]
                rv = r_v[i, pl.ds(j * LANES, LANES)]
                tv = t_v[i, pl.ds(j * LANES, LANES)]
                acc = acc + jnp.abs(hv + rv - tv)
            for k in (1, 2, 4, 8):
                acc = acc + jnp.take(acc, lane ^ k, mode="fill")
            scores = scores - jnp.where(lane == l, acc, 0.0)
        out_v[pl.ds(g * LANES, LANES)] = scores
        return carry

    lax.fori_loop(0, BPW // LANES, group_body, 0)
    pltpu.sync_copy(out_v, out_hbm.at[pl.ds(base, BPW)])


@functools.partial(jax.jit, static_argnames=())
def _sc_score(hidx, ridx, tidx, entity_embedding, relation_embedding):
    mesh = plsc.VectorSubcoreMesh(
        core_axis_name="c", subcore_axis_name="s",
        num_cores=NC, num_subcores=NS)
    run = pl.kernel(
        _sc_body,
        out_type=jax.ShapeDtypeStruct((B,), jnp.float32),
        mesh=mesh,
        scratch_types=[
            pltpu.VMEM((BPW,), jnp.int32),
            pltpu.VMEM((BPW,), jnp.int32),
            pltpu.VMEM((BPW,), jnp.int32),
            pltpu.VMEM((BPW, D), jnp.float32),
            pltpu.VMEM((BPW, D), jnp.float32),
            pltpu.VMEM((BPW, D), jnp.float32),
            pltpu.VMEM((BPW,), jnp.float32),
            pltpu.SemaphoreType.DMA,
            pltpu.SemaphoreType.DMA,
            pltpu.SemaphoreType.DMA,
        ],
    )
    return run(hidx, ridx, tidx, entity_embedding, relation_embedding)


def kernel(sample, entity_embedding, relation_embedding):
    hidx = sample[:, 0]
    ridx = sample[:, 1]
    tidx = sample[:, 2]
    score = _sc_score(hidx, ridx, tidx, entity_embedding, relation_embedding)
    return score[:, None]
